# baseline (speedup 1.0000x reference)
import functools

import jax
import jax.numpy as jnp
from jax import lax
from jax.experimental import pallas as pl
from jax.experimental.pallas import tpu as pltpu
from jax.experimental.pallas import tpu_sc as plsc

N = 10000
E = 160000
D = 256
G = 128
L = 5
T = 1


NC = 2
NS = 16
NW = NC * NS
EB = 128
NBLK = E // EB
_EXTRA = NBLK - (NBLK // NW) * NW

_MESH = plsc.VectorSubcoreMesh(core_axis_name="c", subcore_axis_name="s")


@functools.partial(
    pl.kernel,
    out_type=jax.ShapeDtypeStruct((E, D), jnp.float32),
    mesh=_MESH,
    scratch_types=[
        pltpu.VMEM((EB,), jnp.int32),
        pltpu.VMEM((EB,), jnp.int32),
        pltpu.VMEM((EB, D), jnp.float32),
        pltpu.VMEM((EB, D), jnp.float32),
        pltpu.SemaphoreType.DMA,
        pltpu.SemaphoreType.DMA,
    ],
)
def _msg(h, bond, src, att, out, idx_s, idx_a, hrows, erows, sem1, sem2):
    c = lax.axis_index("c")
    s = lax.axis_index("s")
    wid = s * NC + c
    nb = jnp.where(wid < _EXTRA, NBLK // NW + 1, NBLK // NW)

    def block(k, carry):
        off = (wid + k * NW) * EB
        pltpu.sync_copy(src.at[pl.ds(off, EB)], idx_s)
        pltpu.sync_copy(att.at[pl.ds(off, EB)], idx_a)
        cp1 = pltpu.async_copy(h.at[idx_s], hrows, sem1)
        cp2 = pltpu.async_copy(bond.at[idx_a], erows, sem2)
        cp1.wait()
        cp2.wait()

        def edge(i, c2):
            for j in range(D // 16):
                sl = pl.ds(j * 16, 16)
                hrows[i, sl] = jnp.maximum(hrows[i, sl] + erows[i, sl], 0.0)
            return c2

        lax.fori_loop(0, EB, edge, 0)
        pltpu.sync_copy(hrows, out.at[pl.ds(off, EB)])
        return carry

    lax.fori_loop(0, nb, block, 0)


def _lin1_body(h, agg, eps, w1, b1, o, mo):
    z = (1.0 + eps[0, 0]) * h[...] + agg[...]
    z1 = jnp.dot(z, w1[...]) + b1[...]
    o[...] = z1
    mo[...] = jnp.mean(z1, axis=0, keepdims=True)


def _lin1(h, agg, eps, w1, b1):
    return pl.pallas_call(
        _lin1_body,
        out_shape=(jax.ShapeDtypeStruct((N, D), jnp.float32),
                   jax.ShapeDtypeStruct((1, D), jnp.float32)),
    )(h, agg, eps, w1, b1)


def _lin2_body(z1, m1, s1, g1, bt1, w2, b2, o, mo):
    zn = (z1[...] - m1[...]) / s1[...] * g1[...] + bt1[...]
    zn = jnp.maximum(zn, 0.0)
    z2 = jnp.dot(zn, w2[...]) + b2[...]
    o[...] = z2
    mo[...] = jnp.mean(z2, axis=0, keepdims=True)


def _lin2(z1, m1, s1, g1, bt1, w2, b2):
    return pl.pallas_call(
        _lin2_body,
        out_shape=(jax.ShapeDtypeStruct((N, D), jnp.float32),
                   jax.ShapeDtypeStruct((1, D), jnp.float32)),
    )(z1, m1, s1, g1, bt1, w2, b2)


def _norm_body(z2, m2, s2, g2, bt2, o, *, relu):
    zn = (z2[...] - m2[...]) / s2[...] * g2[...] + bt2[...]
    if relu:
        zn = jnp.maximum(zn, 0.0)
    o[...] = zn


def _norm(z2, m2, s2, g2, bt2, relu):
    return pl.pallas_call(
        functools.partial(_norm_body, relu=relu),
        out_shape=jax.ShapeDtypeStruct((N, D), jnp.float32),
    )(z2, m2, s2, g2, bt2)


def kernel(x, edge_index, edge_attr, batch, params):
    h = params['atom_table'][x]
    src = edge_index[0].astype(jnp.int32)
    att = edge_attr.astype(jnp.int32)
    dst = edge_index[1]
    for l in range(L):
        lp = params['layers'][l]
        msg = _msg(h, params['bond_table'], src, att)
        agg = jnp.zeros_like(h).at[dst].add(msg)
        eps = lp['eps'].reshape(1, 1).astype(jnp.float32)
        z1, m1 = _lin1(h, agg, eps, lp['W1'], lp['b1'].reshape(1, D))
        v1 = jnp.mean((z1 - m1) ** 2, axis=0, keepdims=True)
        s1 = jnp.sqrt(v1 + 1e-5)
        z2, m2 = _lin2(z1, m1, s1,
                       lp['g1'].reshape(1, D), lp['beta1'].reshape(1, D),
                       lp['W2'], lp['b2'].reshape(1, D))
        v2 = jnp.mean((z2 - m2) ** 2, axis=0, keepdims=True)
        s2 = jnp.sqrt(v2 + 1e-5)
        h = _norm(z2, m2, s2,
                  lp['gbn'].reshape(1, D), lp['bbn'].reshape(1, D),
                  relu=(l < L - 1))
    sums = jax.ops.segment_sum(h, batch, num_segments=G)
    cnt = jax.ops.segment_sum(jnp.ones((N, 1), dtype=h.dtype), batch, num_segments=G)
    out = (sums / jnp.maximum(cnt, 1.0)) @ params['pred_W'] + params['pred_b']
    return out.reshape(-1)
